# baseline (speedup 1.0000x reference)
import jax
import jax.numpy as jnp
from jax.experimental import pallas as pl
from jax.experimental.pallas import tpu as pltpu

NSEG = 8
FOLD = 24
SLABS = (0, 3, 21)


def _fam_copies(x, o):
    return [
        (x.at[:, 0:7, 8:24], o.at[:, 1:8, 8:24]),
        (x.at[:, 7:8, 8:24], o.at[:, 0:1, 8:24]),
        (x.at[:, :, 32:168], o.at[:, :, 32:168]),
        (x.at[:, 1:8, 176:197], o.at[:, 0:7, 176:197]),
        (x.at[:, 0:1, 176:197], o.at[:, 7:8, 176:197]),
    ]


def _body(x_hbm, o_hbm, in_buf, out_buf, in_sems, out_sems, fam_sems):
    b = pl.program_id(0)
    nb = pl.num_programs(0)
    slot = jax.lax.rem(b, 2)

    def in_copies(step, slot_):
        return [
            pltpu.make_async_copy(
                x_hbm.at[step, :, tb * 8:(tb + 1) * 8, :],
                in_buf.at[slot_, k],
                in_sems.at[slot_],
            )
            for k, tb in enumerate(SLABS)
        ]

    def out_copies(step, slot_):
        return [
            pltpu.make_async_copy(
                out_buf.at[slot_, k],
                o_hbm.at[step, :, tb * 8:(tb + 1) * 8, :],
                out_sems.at[slot_],
            )
            for k, tb in enumerate(SLABS)
        ]

    @pl.when(b == 0)
    def _():
        for i, (s, d) in enumerate(_fam_copies(x_hbm, o_hbm)):
            pltpu.make_async_copy(s, d, fam_sems.at[i]).start()
        for c in in_copies(0, 0):
            c.start()

    @pl.when(b + 1 < nb)
    def _():
        for c in in_copies(b + 1, 1 - slot):
            c.start()

    for c in in_copies(b, slot):
        c.wait()

    @pl.when(b >= 2)
    def _():
        for c in out_copies(b - 2, slot):
            c.wait()

    xin = in_buf[slot]
    fwd = jnp.concatenate([xin[:, NSEG - 1:], xin[:, :NSEG - 1]], axis=1)
    bwd = jnp.concatenate([xin[:, 1:], xin[:, :1]], axis=1)
    r = jax.lax.broadcasted_iota(jnp.int32, xin.shape[1:], dimension=1)
    out0 = jnp.where(r >= 1, fwd[0], xin[0])
    out1 = jnp.where(r == 0, fwd[1], xin[1])
    out2 = jnp.where(r >= 5, bwd[2], xin[2])
    out_buf[slot] = jnp.stack([out0, out1, out2], axis=0)

    for c in out_copies(b, slot):
        c.start()

    @pl.when(b == nb - 1)
    def _():
        @pl.when(b >= 1)
        def _():
            for c in out_copies(b - 1, 1 - slot):
                c.wait()
        for c in out_copies(b, slot):
            c.wait()
        for i, (s, d) in enumerate(_fam_copies(x_hbm, o_hbm)):
            pltpu.make_async_copy(s, d, fam_sems.at[i]).wait()


def kernel(x):
    nt, l, c = x.shape
    nb = nt // NSEG
    xr = x.reshape(nb, NSEG, l, c)
    out = pl.pallas_call(
        _body,
        grid=(nb,),
        in_specs=[pl.BlockSpec(memory_space=pltpu.MemorySpace.HBM)],
        out_specs=pl.BlockSpec(memory_space=pltpu.MemorySpace.HBM),
        out_shape=jax.ShapeDtypeStruct((nb, NSEG, l, c), x.dtype),
        scratch_shapes=[
            pltpu.VMEM((2, 3, NSEG, 8, c), x.dtype),
            pltpu.VMEM((2, 3, NSEG, 8, c), x.dtype),
            pltpu.SemaphoreType.DMA((2,)),
            pltpu.SemaphoreType.DMA((2,)),
            pltpu.SemaphoreType.DMA((5,)),
        ],
    )(xr)
    return out.reshape(nt, l, c)
